# baseline (speedup 1.0000x reference)
import functools

import jax
import jax.numpy as jnp
from jax import lax
from jax.experimental import pallas as pl
from jax.experimental.pallas import tpu as pltpu
from jax.experimental.pallas import tpu_sc as plsc

NB = 30
B = 4

_SC_NC = 2
_SC_NS = 16
_SC_NW = _SC_NC * _SC_NS
_SC_L = 16


def _sc_level(m_hbm, f_buf, mb, sems, acc_ref, r0, rows, n0, nn):
    z = jnp.zeros((_SC_L,), jnp.float32)
    for b in range(B):
        for n in range(32):
            acc_ref[b, n] = z

    pltpu.make_async_copy(
        m_hbm.at[n0, pl.ds(r0, rows), :], mb[0], sems[0]
    ).start()

    def n_body(i, _):
        for p in (0, 1):

            @pl.when(i % 2 == p)
            def _():
                pltpu.make_async_copy(
                    m_hbm.at[n0 + i, pl.ds(r0, rows), :], mb[p], sems[p]
                ).wait()

                @pl.when(i + 1 < nn)
                def _():
                    pltpu.make_async_copy(
                        m_hbm.at[n0 + i + 1, pl.ds(r0, rows), :],
                        mb[1 - p],
                        sems[1 - p],
                    ).start()

                accs0 = tuple(z for _ in range(B))

                def r_body(r, accs):
                    out = list(accs)
                    for cc in range(128 // _SC_L):
                        off = cc * _SC_L
                        mv = mb[p][r, pl.ds(off, _SC_L)]
                        for b in range(B):
                            d = mv - f_buf[b, r, pl.ds(off, _SC_L)]
                            out[b] = out[b] + d * d
                    return tuple(out)

                accs = lax.fori_loop(0, rows, r_body, accs0)
                for b in range(B):
                    acc_ref[b, n0 + i] = accs[b]

        return 0

    lax.fori_loop(0, nn, n_body, 0)


def _sc_dist_body(m1_hbm, f1_hbm, m2_hbm, f2_hbm, out_hbm,
                  f1_buf, f2_buf, m1a, m1b, m2a, m2b, acc_ref,
                  s1a, s1b, s2a, s2b):
    rows1 = m1_hbm.shape[1] // _SC_NW
    rows2 = 2 * (m2_hbm.shape[1] // _SC_NW)
    wid = lax.axis_index("s") * _SC_NC + lax.axis_index("c")
    r01 = wid * rows1
    r02 = (wid // 2) * rows2
    n02 = (wid % 2) * (NB // 2)

    pltpu.sync_copy(f1_hbm.at[:, pl.ds(r01, rows1), :], f1_buf)
    pltpu.sync_copy(f2_hbm.at[:, pl.ds(r02, rows2), :], f2_buf)

    _sc_level(m1_hbm, f1_buf, (m1a, m1b), (s1a, s1b), acc_ref,
              r01, rows1, 0, NB)
    pltpu.sync_copy(acc_ref, out_hbm.at[0, wid])

    _sc_level(m2_hbm, f2_buf, (m2a, m2b), (s2a, s2b), acc_ref,
              r02, rows2, n02, NB // 2)
    pltpu.sync_copy(acc_ref, out_hbm.at[1, wid])


def _sc_dist(m1r, f1r, m2r, f2r):
    rows1 = m1r.shape[1] // _SC_NW
    rows2 = 2 * (m2r.shape[1] // _SC_NW)
    mesh = plsc.VectorSubcoreMesh(core_axis_name="c", subcore_axis_name="s")
    f = pl.kernel(
        _sc_dist_body,
        mesh=mesh,
        out_type=jax.ShapeDtypeStruct((2, _SC_NW, B, 32, _SC_L), jnp.float32),
        scratch_types=[
            pltpu.VMEM((B, rows1, 128), jnp.float32),
            pltpu.VMEM((B, rows2, 128), jnp.float32),
            pltpu.VMEM((rows1, 128), jnp.float32),
            pltpu.VMEM((rows1, 128), jnp.float32),
            pltpu.VMEM((rows2, 128), jnp.float32),
            pltpu.VMEM((rows2, 128), jnp.float32),
            pltpu.VMEM((B, 32, _SC_L), jnp.float32),
            pltpu.SemaphoreType.DMA,
            pltpu.SemaphoreType.DMA,
            pltpu.SemaphoreType.DMA,
            pltpu.SemaphoreType.DMA,
        ],
    )
    return f(m1r, f1r, m2r, f2r)


def _dist_body(f_ref, m_ref, o_ref):
    step = pl.program_id(0)

    @pl.when(step == 0)
    def _():
        o_ref[...] = jnp.zeros_like(o_ref)

    m = m_ref[...]
    for b in range(B):
        d = m - f_ref[b : b + 1]
        o_ref[b, :] += jnp.sum(d * d, axis=(1, 2, 3))


def _dist(f, m, cc):
    _, c, h, w = f.shape
    n = c // cc
    return pl.pallas_call(
        _dist_body,
        grid=(n,),
        in_specs=[
            pl.BlockSpec((B, cc, h, w), lambda i: (0, i, 0, 0)),
            pl.BlockSpec((NB, cc, h, w), lambda i: (0, i, 0, 0)),
        ],
        out_specs=pl.BlockSpec((B, NB), lambda i: (0, 0)),
        out_shape=jax.ShapeDtypeStruct((B, NB), jnp.float32),
    )(f, m)


def _argmin_body(s0_ref, sc_ref, o_ref, *, scales):
    s0 = s0_ref[...]
    red = jnp.sum(sc_ref[...], axis=(1, 4))
    s = (
        s0 * scales[0]
        + red[0][:, :NB] * scales[1]
        + red[1][:, :NB] * scales[2]
    )
    mn = jnp.min(s, axis=1, keepdims=True)
    ii = jax.lax.broadcasted_iota(jnp.int32, s.shape, 1)
    cand = jnp.where(s == mn, ii, NB)
    idx = jnp.min(cand, axis=1, keepdims=True)
    o_ref[...] = jnp.concatenate(
        [idx, jnp.zeros((8 - B, 1), jnp.int32)], axis=0
    )


def _argmin(s0, sc_out, scales):
    out = pl.pallas_call(
        functools.partial(_argmin_body, scales=scales),
        out_shape=jax.ShapeDtypeStruct((8, 1), jnp.int32),
    )(s0, sc_out)
    return out.reshape(8)


def _sc_gather_unit(fr, mr, s, stage, o_hbm, h, b, r0_rows, level):
    pltpu.sync_copy(fr.at[b, pl.ds(h * 48, 48), :], r0_rows[0])
    pltpu.sync_copy(mr.at[s, pl.ds(h * 48, 48), :], r0_rows[1])

    def r_body(r, _):
        if level == 1:
            wt = r // 8
            w8 = r % 8
            tf = wt * 16 + w8
            td = tf + 8
        else:
            w = r // 2
            ci = r % 2
            wt = w // 8
            w8 = w % 8
            tf = wt * 32 + ci * 8 + w8
            td = tf + 16
        for cc in range(128 // _SC_L):
            off = cc * _SC_L
            fv = r0_rows[0][r, pl.ds(off, _SC_L)]
            mv = r0_rows[1][r, pl.ds(off, _SC_L)]
            d = mv - fv
            stage[tf, pl.ds(off, _SC_L)] = fv
            stage[td, pl.ds(off, _SC_L)] = d * d
        return 0

    lax.fori_loop(0, 48, r_body, 0)
    pltpu.sync_copy(stage, o_hbm.at[b, h])


def _sc_gather_body(idx_hbm, m1_hbm, f1_hbm, m2_hbm, f2_hbm,
                    o1_hbm, o2_hbm, idx_v, fbuf, mbuf, stage):
    wid = lax.axis_index("s") * _SC_NC + lax.axis_index("c")
    pltpu.sync_copy(idx_hbm, idx_v.at[pl.ds(0, 8)])
    iv = idx_v[...]

    @pl.when(wid < 24)
    def _():
        def h_body(k, _):
            h = wid * 2 + k
            for b in range(B):
                _sc_gather_unit(f1_hbm, m1_hbm, iv[b], stage,
                                o1_hbm, h, b, (fbuf, mbuf), 1)
            return 0

        lax.fori_loop(0, 2, h_body, 0)

    @pl.when(wid >= 24)
    def _():
        def h_body(k, _):
            h = (wid - 24) * 3 + k
            for b in range(B):
                _sc_gather_unit(f2_hbm, m2_hbm, iv[b], stage,
                                o2_hbm, h, b, (fbuf, mbuf), 2)
            return 0

        lax.fori_loop(0, 3, h_body, 0)


def _sc_gather(idx8, m1r, f1r, m2r, f2r, h1, h2):
    mesh = plsc.VectorSubcoreMesh(core_axis_name="c", subcore_axis_name="s")
    f = pl.kernel(
        _sc_gather_body,
        mesh=mesh,
        out_type=(
            jax.ShapeDtypeStruct((B, h1, 96, 128), jnp.float32),
            jax.ShapeDtypeStruct((B, h2, 96, 128), jnp.float32),
        ),
        scratch_types=[
            pltpu.VMEM((16,), jnp.int32),
            pltpu.VMEM((48, 128), jnp.float32),
            pltpu.VMEM((48, 128), jnp.float32),
            pltpu.VMEM((96, 128), jnp.float32),
        ],
    )
    return f(idx8, m1r, f1r, m2r, f2r)


def _gather_cfirst_body(idx_ref, f_ref, m_ref, o_ref):
    f = f_ref[0]
    m = m_ref[0]
    o_ref[0, 0] = f
    d = m - f
    o_ref[0, 1] = d * d


def _gather_cfirst(idx, f, m, cc):
    _, c, h, w = f.shape
    nc = c // cc
    grid_spec = pltpu.PrefetchScalarGridSpec(
        num_scalar_prefetch=1,
        grid=(B, nc),
        in_specs=[
            pl.BlockSpec((1, cc, h, w), lambda b, i, idx_ref: (b, i, 0, 0)),
            pl.BlockSpec(
                (1, cc, h, w), lambda b, i, idx_ref: (idx_ref[b], i, 0, 0)
            ),
        ],
        out_specs=pl.BlockSpec(
            (1, 2, cc, h, w), lambda b, i, idx_ref: (b, 0, i, 0, 0)
        ),
    )
    out = pl.pallas_call(
        _gather_cfirst_body,
        grid_spec=grid_spec,
        out_shape=jax.ShapeDtypeStruct((B, 2, c, h, w), jnp.float32),
    )(idx, f, m)
    return out.reshape(B, 2 * c, h, w)


def _gather_clast_body(idx_ref, f_ref, m_ref, o_ref, *, nct):
    f3 = f_ref[0]
    m3 = m_ref[0]
    hh, w, c = f3.shape
    fr = f3.reshape(hh, w // 8, 8, c)
    d = m3 - f3
    dr = (d * d).reshape(hh, w // 8, 8, c)
    for t in range(nct):
        o_ref[0, :, :, t] = fr[..., t * 128 : (t + 1) * 128]
        o_ref[0, :, :, nct + t] = dr[..., t * 128 : (t + 1) * 128]


def _gather_clast(idx, fv, mv, hh):
    _, h, w, c = fv.shape
    nct = c // 128
    nh = h // hh
    grid_spec = pltpu.PrefetchScalarGridSpec(
        num_scalar_prefetch=1,
        grid=(B, nh),
        in_specs=[
            pl.BlockSpec((1, hh, w, c), lambda b, i, idx_ref: (b, i, 0, 0)),
            pl.BlockSpec(
                (1, hh, w, c), lambda b, i, idx_ref: (idx_ref[b], i, 0, 0)
            ),
        ],
        out_specs=pl.BlockSpec(
            (1, hh, w // 8, 2 * nct, 8, 128),
            lambda b, i, idx_ref: (b, i, 0, 0, 0, 0),
        ),
    )
    out = pl.pallas_call(
        functools.partial(_gather_clast_body, nct=nct),
        grid_spec=grid_spec,
        out_shape=jax.ShapeDtypeStruct(
            (B, h, w // 8, 2 * nct, 8, 128), jnp.float32
        ),
    )(idx, fv, mv)
    return (
        out.transpose(0, 3, 5, 1, 2, 4).reshape(B, 2 * c, h, w)
    )


_DIST_CC = {0: 4, 1: 8, 2: 8}
_GATHER_L0_CC = 16
_GATHER_HH = {1: 12, 2: 12}


def kernel(features_l0, features_l1, features_l2, mem_l0, mem_l1, mem_l2):
    f1v = features_l1.transpose(0, 2, 3, 1)
    m1v = mem_l1.transpose(0, 2, 3, 1)
    f2v = features_l2.transpose(0, 2, 3, 1)
    m2v = mem_l2.transpose(0, 2, 3, 1)

    r1 = m1v.shape[1] * m1v.shape[2] * (m1v.shape[3] // 128)
    r2 = m2v.shape[1] * m2v.shape[2] * (m2v.shape[3] // 128)
    sc_out = _sc_dist(
        m1v.reshape(NB, r1, 128),
        f1v.reshape(B, r1, 128),
        m2v.reshape(NB, r2, 128),
        f2v.reshape(B, r2, 128),
    )

    s0 = _dist(features_l0, mem_l0, _DIST_CC[0])

    scales = []
    for f in (features_l0, features_l1, features_l2):
        scales.append(1.0 / (f.shape[1] * f.shape[2] * f.shape[3]))
    idx8 = _argmin(s0, sc_out, tuple(scales))
    idx = idx8[:B]

    h1 = f1v.shape[1]
    h2 = f2v.shape[1]
    go1, go2 = _sc_gather(
        idx8,
        m1v.reshape(NB, r1, 128),
        f1v.reshape(B, r1, 128),
        m2v.reshape(NB, r2, 128),
        f2v.reshape(B, r2, 128),
        h1,
        h2,
    )
    o0 = _gather_cfirst(idx, features_l0, mem_l0, _GATHER_L0_CC)
    c1 = features_l1.shape[1]
    c2 = features_l2.shape[1]
    o1 = (
        go1.reshape(B, h1, 6, 2, 8, 128)
        .transpose(0, 3, 5, 1, 2, 4)
        .reshape(B, 2 * c1, h1, h1)
    )
    o2 = (
        go2.reshape(B, h2, 3, 4, 8, 128)
        .transpose(0, 3, 5, 1, 2, 4)
        .reshape(B, 2 * c2, h2, h2)
    )
    return (o0, o1, o2)


# speedup vs baseline: 1.2935x; 1.2935x over previous
import functools

import jax
import jax.numpy as jnp
from jax import lax
from jax.experimental import pallas as pl
from jax.experimental.pallas import tpu as pltpu
from jax.experimental.pallas import tpu_sc as plsc

NB = 30
B = 4

_SC_NC = 2
_SC_NS = 16
_SC_NW = _SC_NC * _SC_NS
_SC_L = 16


def _sc_level(m_hbm, f_buf, mb, sems, acc_ref, r0, rows, n0, nn):
    z = jnp.zeros((_SC_L,), jnp.float32)
    for b in range(B):
        for n in range(32):
            acc_ref[b, n] = z

    pltpu.make_async_copy(
        m_hbm.at[n0, pl.ds(r0, rows), :], mb[0], sems[0]
    ).start()

    def n_body(i, _):
        for p in (0, 1):

            @pl.when(i % 2 == p)
            def _():
                pltpu.make_async_copy(
                    m_hbm.at[n0 + i, pl.ds(r0, rows), :], mb[p], sems[p]
                ).wait()

                @pl.when(i + 1 < nn)
                def _():
                    pltpu.make_async_copy(
                        m_hbm.at[n0 + i + 1, pl.ds(r0, rows), :],
                        mb[1 - p],
                        sems[1 - p],
                    ).start()

                accs0 = tuple(z for _ in range(B))

                def r_body(r, accs):
                    out = list(accs)
                    for cc in range(128 // _SC_L):
                        off = cc * _SC_L
                        mv = mb[p][r, pl.ds(off, _SC_L)]
                        for b in range(B):
                            d = mv - f_buf[b, r, pl.ds(off, _SC_L)]
                            out[b] = out[b] + d * d
                    return tuple(out)

                accs = lax.fori_loop(0, rows, r_body, accs0)
                for b in range(B):
                    acc_ref[b, n0 + i] = accs[b]

        return 0

    lax.fori_loop(0, nn, n_body, 0)


def _sc_dist_body(m1_hbm, f1_hbm, m2_hbm, f2_hbm, out_hbm,
                  f1_buf, f2_buf, m1a, m1b, m2a, m2b, acc_ref,
                  s1a, s1b, s2a, s2b):
    rows1 = m1_hbm.shape[1] // _SC_NW
    rows2 = 2 * (m2_hbm.shape[1] // _SC_NW)
    wid = lax.axis_index("s") * _SC_NC + lax.axis_index("c")
    r01 = wid * rows1
    r02 = (wid // 2) * rows2
    n02 = (wid % 2) * (NB // 2)

    pltpu.sync_copy(f1_hbm.at[:, pl.ds(r01, rows1), :], f1_buf)
    pltpu.sync_copy(f2_hbm.at[:, pl.ds(r02, rows2), :], f2_buf)

    _sc_level(m1_hbm, f1_buf, (m1a, m1b), (s1a, s1b), acc_ref,
              r01, rows1, 0, NB)
    pltpu.sync_copy(acc_ref, out_hbm.at[0, wid])

    _sc_level(m2_hbm, f2_buf, (m2a, m2b), (s2a, s2b), acc_ref,
              r02, rows2, n02, NB // 2)
    pltpu.sync_copy(acc_ref, out_hbm.at[1, wid])


def _sc_dist(m1r, f1r, m2r, f2r):
    rows1 = m1r.shape[1] // _SC_NW
    rows2 = 2 * (m2r.shape[1] // _SC_NW)
    mesh = plsc.VectorSubcoreMesh(core_axis_name="c", subcore_axis_name="s")
    f = pl.kernel(
        _sc_dist_body,
        mesh=mesh,
        out_type=jax.ShapeDtypeStruct((2, _SC_NW, B, 32, _SC_L), jnp.float32),
        scratch_types=[
            pltpu.VMEM((B, rows1, 128), jnp.float32),
            pltpu.VMEM((B, rows2, 128), jnp.float32),
            pltpu.VMEM((rows1, 128), jnp.float32),
            pltpu.VMEM((rows1, 128), jnp.float32),
            pltpu.VMEM((rows2, 128), jnp.float32),
            pltpu.VMEM((rows2, 128), jnp.float32),
            pltpu.VMEM((B, 32, _SC_L), jnp.float32),
            pltpu.SemaphoreType.DMA,
            pltpu.SemaphoreType.DMA,
            pltpu.SemaphoreType.DMA,
            pltpu.SemaphoreType.DMA,
        ],
    )
    return f(m1r, f1r, m2r, f2r)


def _dist_body(f_ref, m_ref, o_ref):
    step = pl.program_id(0)

    @pl.when(step == 0)
    def _():
        o_ref[...] = jnp.zeros_like(o_ref)

    m = m_ref[...]
    for b in range(B):
        d = m - f_ref[b : b + 1]
        o_ref[b, :] += jnp.sum(d * d, axis=(1, 2, 3))


def _dist(f, m, cc):
    _, c, h, w = f.shape
    n = c // cc
    return pl.pallas_call(
        _dist_body,
        grid=(n,),
        in_specs=[
            pl.BlockSpec((B, cc, h, w), lambda i: (0, i, 0, 0)),
            pl.BlockSpec((NB, cc, h, w), lambda i: (0, i, 0, 0)),
        ],
        out_specs=pl.BlockSpec((B, NB), lambda i: (0, 0)),
        out_shape=jax.ShapeDtypeStruct((B, NB), jnp.float32),
    )(f, m)


def _argmin_body(s0_ref, sc_ref, o_ref, *, scales):
    s0 = s0_ref[...]
    red = jnp.sum(sc_ref[...], axis=(1, 4))
    s = (
        s0 * scales[0]
        + red[0][:, :NB] * scales[1]
        + red[1][:, :NB] * scales[2]
    )
    mn = jnp.min(s, axis=1, keepdims=True)
    ii = jax.lax.broadcasted_iota(jnp.int32, s.shape, 1)
    cand = jnp.where(s == mn, ii, NB)
    o_ref[...] = jnp.min(cand, axis=1, keepdims=True)


def _argmin(s0, sc_out, scales):
    out = pl.pallas_call(
        functools.partial(_argmin_body, scales=scales),
        out_shape=jax.ShapeDtypeStruct((B, 1), jnp.int32),
    )(s0, sc_out)
    return out.reshape(B)


def _gather_cfirst_body(idx_ref, f_ref, m_ref, o_ref):
    f = f_ref[0]
    m = m_ref[0]
    o_ref[0, 0] = f
    d = m - f
    o_ref[0, 1] = d * d


def _gather_cfirst(idx, f, m, cc):
    _, c, h, w = f.shape
    nc = c // cc
    grid_spec = pltpu.PrefetchScalarGridSpec(
        num_scalar_prefetch=1,
        grid=(B, nc),
        in_specs=[
            pl.BlockSpec((1, cc, h, w), lambda b, i, idx_ref: (b, i, 0, 0)),
            pl.BlockSpec(
                (1, cc, h, w), lambda b, i, idx_ref: (idx_ref[b], i, 0, 0)
            ),
        ],
        out_specs=pl.BlockSpec(
            (1, 2, cc, h, w), lambda b, i, idx_ref: (b, 0, i, 0, 0)
        ),
    )
    out = pl.pallas_call(
        _gather_cfirst_body,
        grid_spec=grid_spec,
        out_shape=jax.ShapeDtypeStruct((B, 2, c, h, w), jnp.float32),
    )(idx, f, m)
    return out.reshape(B, 2 * c, h, w)


def _gather_clast_body(idx_ref, f_ref, m_ref, o_ref, *, nct):
    f3 = f_ref[0]
    m3 = m_ref[0]
    hh, w, c = f3.shape
    fr = f3.reshape(hh, w // 8, 8, c)
    d = m3 - f3
    dr = (d * d).reshape(hh, w // 8, 8, c)
    for t in range(nct):
        o_ref[0, :, :, t] = fr[..., t * 128 : (t + 1) * 128]
        o_ref[0, :, :, nct + t] = dr[..., t * 128 : (t + 1) * 128]


def _gather_clast(idx, fv, mv, hh):
    _, h, w, c = fv.shape
    nct = c // 128
    nh = h // hh
    grid_spec = pltpu.PrefetchScalarGridSpec(
        num_scalar_prefetch=1,
        grid=(B, nh),
        in_specs=[
            pl.BlockSpec((1, hh, w, c), lambda b, i, idx_ref: (b, i, 0, 0)),
            pl.BlockSpec(
                (1, hh, w, c), lambda b, i, idx_ref: (idx_ref[b], i, 0, 0)
            ),
        ],
        out_specs=pl.BlockSpec(
            (1, hh, w // 8, 2 * nct, 8, 128),
            lambda b, i, idx_ref: (b, i, 0, 0, 0, 0),
        ),
    )
    out = pl.pallas_call(
        functools.partial(_gather_clast_body, nct=nct),
        grid_spec=grid_spec,
        out_shape=jax.ShapeDtypeStruct(
            (B, h, w // 8, 2 * nct, 8, 128), jnp.float32
        ),
    )(idx, fv, mv)
    return (
        out.transpose(0, 3, 5, 1, 2, 4).reshape(B, 2 * c, h, w)
    )


_DIST_CC = {0: 8, 1: 8, 2: 8}
_GATHER_L0_CC = 16
_GATHER_HH = {1: 12, 2: 12}


def kernel(features_l0, features_l1, features_l2, mem_l0, mem_l1, mem_l2):
    f1v = features_l1.transpose(0, 2, 3, 1)
    m1v = mem_l1.transpose(0, 2, 3, 1)
    f2v = features_l2.transpose(0, 2, 3, 1)
    m2v = mem_l2.transpose(0, 2, 3, 1)

    r1 = m1v.shape[1] * m1v.shape[2] * (m1v.shape[3] // 128)
    r2 = m2v.shape[1] * m2v.shape[2] * (m2v.shape[3] // 128)
    sc_out = _sc_dist(
        m1v.reshape(NB, r1, 128),
        f1v.reshape(B, r1, 128),
        m2v.reshape(NB, r2, 128),
        f2v.reshape(B, r2, 128),
    )

    s0 = _dist(features_l0, mem_l0, _DIST_CC[0])

    scales = []
    for f in (features_l0, features_l1, features_l2):
        scales.append(1.0 / (f.shape[1] * f.shape[2] * f.shape[3]))
    idx = _argmin(s0, sc_out, tuple(scales))

    o0 = _gather_cfirst(idx, features_l0, mem_l0, _GATHER_L0_CC)
    o1 = _gather_clast(idx, f1v, m1v, _GATHER_HH[1])
    o2 = _gather_clast(idx, f2v, m2v, _GATHER_HH[2])
    return (o0, o1, o2)


# speedup vs baseline: 1.3035x; 1.0078x over previous
import functools

import jax
import jax.numpy as jnp
from jax import lax
from jax.experimental import pallas as pl
from jax.experimental.pallas import tpu as pltpu
from jax.experimental.pallas import tpu_sc as plsc

NB = 30
B = 4

_SC_NC = 2
_SC_NS = 16
_SC_NW = _SC_NC * _SC_NS
_SC_L = 16


def _sc_level(m_hbm, f_buf, mb, sems, acc_ref, r0, rows, n0, nn):
    z = jnp.zeros((_SC_L,), jnp.float32)
    for b in range(B):
        for n in range(32):
            acc_ref[b, n] = z

    pltpu.make_async_copy(
        m_hbm.at[n0, pl.ds(r0, rows), :], mb[0], sems[0]
    ).start()

    def n_body(i, _):
        for p in (0, 1):

            @pl.when(i % 2 == p)
            def _():
                pltpu.make_async_copy(
                    m_hbm.at[n0 + i, pl.ds(r0, rows), :], mb[p], sems[p]
                ).wait()

                @pl.when(i + 1 < nn)
                def _():
                    pltpu.make_async_copy(
                        m_hbm.at[n0 + i + 1, pl.ds(r0, rows), :],
                        mb[1 - p],
                        sems[1 - p],
                    ).start()

                accs0 = tuple(z for _ in range(B))

                def r_body(r, accs):
                    out = list(accs)
                    for cc in range(128 // _SC_L):
                        off = cc * _SC_L
                        mv = mb[p][r, pl.ds(off, _SC_L)]
                        for b in range(B):
                            d = mv - f_buf[b, r, pl.ds(off, _SC_L)]
                            out[b] = out[b] + d * d
                    return tuple(out)

                accs = lax.fori_loop(0, rows, r_body, accs0)
                for b in range(B):
                    acc_ref[b, n0 + i] = accs[b]

        return 0

    lax.fori_loop(0, nn, n_body, 0)


def _sc_dist_body(m1_hbm, f1_hbm, m2_hbm, f2_hbm, out_hbm,
                  f1_buf, f2_buf, m1a, m1b, m2a, m2b, acc_ref,
                  s1a, s1b, s2a, s2b):
    rows1 = m1_hbm.shape[1] // _SC_NW
    rows2 = 2 * (m2_hbm.shape[1] // _SC_NW)
    wid = lax.axis_index("s") * _SC_NC + lax.axis_index("c")
    r01 = wid * rows1
    r02 = (wid // 2) * rows2
    n02 = (wid % 2) * (NB // 2)

    pltpu.sync_copy(f1_hbm.at[:, pl.ds(r01, rows1), :], f1_buf)
    pltpu.sync_copy(f2_hbm.at[:, pl.ds(r02, rows2), :], f2_buf)

    _sc_level(m1_hbm, f1_buf, (m1a, m1b), (s1a, s1b), acc_ref,
              r01, rows1, 0, NB)
    pltpu.sync_copy(acc_ref, out_hbm.at[0, wid])

    _sc_level(m2_hbm, f2_buf, (m2a, m2b), (s2a, s2b), acc_ref,
              r02, rows2, n02, NB // 2)
    pltpu.sync_copy(acc_ref, out_hbm.at[1, wid])


def _sc_dist(m1r, f1r, m2r, f2r):
    rows1 = m1r.shape[1] // _SC_NW
    rows2 = 2 * (m2r.shape[1] // _SC_NW)
    mesh = plsc.VectorSubcoreMesh(core_axis_name="c", subcore_axis_name="s")
    f = pl.kernel(
        _sc_dist_body,
        mesh=mesh,
        out_type=jax.ShapeDtypeStruct((2, _SC_NW, B, 32, _SC_L), jnp.float32),
        scratch_types=[
            pltpu.VMEM((B, rows1, 128), jnp.float32),
            pltpu.VMEM((B, rows2, 128), jnp.float32),
            pltpu.VMEM((rows1, 128), jnp.float32),
            pltpu.VMEM((rows1, 128), jnp.float32),
            pltpu.VMEM((rows2, 128), jnp.float32),
            pltpu.VMEM((rows2, 128), jnp.float32),
            pltpu.VMEM((B, 32, _SC_L), jnp.float32),
            pltpu.SemaphoreType.DMA,
            pltpu.SemaphoreType.DMA,
            pltpu.SemaphoreType.DMA,
            pltpu.SemaphoreType.DMA,
        ],
    )
    return f(m1r, f1r, m2r, f2r)


def _dist_body(f_ref, m_ref, o_ref):
    step = pl.program_id(0)

    @pl.when(step == 0)
    def _():
        o_ref[...] = jnp.zeros_like(o_ref)

    m = m_ref[...]
    for b in range(B):
        d = m - f_ref[b : b + 1]
        o_ref[b, :] += jnp.sum(d * d, axis=(1, 2, 3))


def _dist(f, m, cc):
    _, c, h, w = f.shape
    n = c // cc
    return pl.pallas_call(
        _dist_body,
        grid=(n,),
        in_specs=[
            pl.BlockSpec((B, cc, h, w), lambda i: (0, i, 0, 0)),
            pl.BlockSpec((NB, cc, h, w), lambda i: (0, i, 0, 0)),
        ],
        out_specs=pl.BlockSpec((B, NB), lambda i: (0, 0)),
        out_shape=jax.ShapeDtypeStruct((B, NB), jnp.float32),
    )(f, m)


def _argmin_body(s0_ref, sc_ref, o_ref, *, scales):
    s0 = s0_ref[...]
    red = jnp.sum(sc_ref[...], axis=(1, 4))
    s = (
        s0 * scales[0]
        + red[0][:, :NB] * scales[1]
        + red[1][:, :NB] * scales[2]
    )
    mn = jnp.min(s, axis=1, keepdims=True)
    ii = jax.lax.broadcasted_iota(jnp.int32, s.shape, 1)
    cand = jnp.where(s == mn, ii, NB)
    o_ref[...] = jnp.min(cand, axis=1, keepdims=True)


def _argmin(s0, sc_out, scales):
    out = pl.pallas_call(
        functools.partial(_argmin_body, scales=scales),
        out_shape=jax.ShapeDtypeStruct((B, 1), jnp.int32),
    )(s0, sc_out)
    return out.reshape(B)


def _gather_cfirst_body(idx_ref, f_ref, m_ref, o_ref):
    f = f_ref[0]
    m = m_ref[0]
    o_ref[0, 0] = f
    d = m - f
    o_ref[0, 1] = d * d


def _gather_cfirst(idx, f, m, cc):
    _, c, h, w = f.shape
    nc = c // cc
    grid_spec = pltpu.PrefetchScalarGridSpec(
        num_scalar_prefetch=1,
        grid=(B, nc),
        in_specs=[
            pl.BlockSpec((1, cc, h, w), lambda b, i, idx_ref: (b, i, 0, 0)),
            pl.BlockSpec(
                (1, cc, h, w), lambda b, i, idx_ref: (idx_ref[b], i, 0, 0)
            ),
        ],
        out_specs=pl.BlockSpec(
            (1, 2, cc, h, w), lambda b, i, idx_ref: (b, 0, i, 0, 0)
        ),
    )
    out = pl.pallas_call(
        _gather_cfirst_body,
        grid_spec=grid_spec,
        out_shape=jax.ShapeDtypeStruct((B, 2, c, h, w), jnp.float32),
    )(idx, f, m)
    return out.reshape(B, 2 * c, h, w)


def _gather_clast_body(idx_ref, f_ref, m_ref, o_ref, *, nct):
    f3 = f_ref[0]
    m3 = m_ref[0]
    hh, w, c = f3.shape
    fr = f3.reshape(hh, w // 8, 8, c)
    d = m3 - f3
    dr = (d * d).reshape(hh, w // 8, 8, c)
    for t in range(nct):
        o_ref[0, :, :, t] = fr[..., t * 128 : (t + 1) * 128]
        o_ref[0, :, :, nct + t] = dr[..., t * 128 : (t + 1) * 128]


def _gather_clast(idx, fv, mv, hh):
    _, h, w, c = fv.shape
    nct = c // 128
    nh = h // hh
    grid_spec = pltpu.PrefetchScalarGridSpec(
        num_scalar_prefetch=1,
        grid=(B, nh),
        in_specs=[
            pl.BlockSpec((1, hh, w, c), lambda b, i, idx_ref: (b, i, 0, 0)),
            pl.BlockSpec(
                (1, hh, w, c), lambda b, i, idx_ref: (idx_ref[b], i, 0, 0)
            ),
        ],
        out_specs=pl.BlockSpec(
            (1, hh, w // 8, 2 * nct, 8, 128),
            lambda b, i, idx_ref: (b, i, 0, 0, 0, 0),
        ),
    )
    out = pl.pallas_call(
        functools.partial(_gather_clast_body, nct=nct),
        grid_spec=grid_spec,
        out_shape=jax.ShapeDtypeStruct(
            (B, h, w // 8, 2 * nct, 8, 128), jnp.float32
        ),
    )(idx, fv, mv)
    return (
        out.transpose(0, 3, 5, 1, 2, 4).reshape(B, 2 * c, h, w)
    )


_DIST_CC = {0: 4, 1: 8, 2: 8}
_GATHER_L0_CC = 16
_GATHER_HH = {1: 12, 2: 12}


def kernel(features_l0, features_l1, features_l2, mem_l0, mem_l1, mem_l2):
    f1v = features_l1.transpose(0, 2, 3, 1)
    m1v = mem_l1.transpose(0, 2, 3, 1)
    f2v = features_l2.transpose(0, 2, 3, 1)
    m2v = mem_l2.transpose(0, 2, 3, 1)

    r1 = m1v.shape[1] * m1v.shape[2] * (m1v.shape[3] // 128)
    r2 = m2v.shape[1] * m2v.shape[2] * (m2v.shape[3] // 128)
    sc_out = _sc_dist(
        m1v.reshape(NB, r1, 128),
        f1v.reshape(B, r1, 128),
        m2v.reshape(NB, r2, 128),
        f2v.reshape(B, r2, 128),
    )

    s0 = _dist(features_l0, mem_l0, _DIST_CC[0])

    scales = []
    for f in (features_l0, features_l1, features_l2):
        scales.append(1.0 / (f.shape[1] * f.shape[2] * f.shape[3]))
    idx = _argmin(s0, sc_out, tuple(scales))

    o0 = _gather_cfirst(idx, features_l0, mem_l0, _GATHER_L0_CC)
    o1 = _gather_clast(idx, f1v, m1v, _GATHER_HH[1])
    o2 = _gather_clast(idx, f2v, m2v, _GATHER_HH[2])
    return (o0, o1, o2)
